# baseline (speedup 1.0000x reference)
import functools

import jax
import jax.numpy as jnp
from jax import lax
from jax.experimental import pallas as pl
from jax.experimental.pallas import tpu as pltpu
from jax.experimental.pallas import tpu_sc as plsc

B, S, D = 2, 2048, 768
E, H, TOP_K, TAU = 8, 2048, 2, 1.0
T = B * S
TM = 512
P = 2 * T + E * TM
NT = P // TM
CHUNK = 512
NCH = T // CHUNK
CB = 1024


def _route_body(x_ref, gw_ref, gb_ref, gum_ref,
                d1_ref, d2_ref, wa_ref, wb_ref, meta_ref):
    x = x_ref[...]
    logits = jnp.dot(x, gw_ref[...], preferred_element_type=jnp.float32)
    noisy = logits + gb_ref[...] + gum_ref[...]

    v1 = jnp.max(noisy, axis=-1)
    i1 = jnp.argmax(noisy, axis=-1)
    cols = jax.lax.broadcasted_iota(jnp.int32, noisy.shape, 1)
    masked = jnp.where(cols == i1[:, None], -jnp.inf, noisy)
    v2 = jnp.max(masked, axis=-1)
    i2 = jnp.argmax(masked, axis=-1)
    tso = jnp.exp(v2 - v1)
    wa_ref[...] = (1.0 / (1.0 + tso))[:, None]
    wb_ref[...] = (tso / (1.0 + tso))[:, None]

    sel1 = cols == i1[:, None]
    sel2 = cols == i2[:, None]
    ind = jnp.where(jnp.logical_or(sel1, sel2), 1.0, 0.0)

    r = jax.lax.broadcasted_iota(jnp.int32, (CHUNK, CHUNK), 0)
    c = jax.lax.broadcasted_iota(jnp.int32, (CHUNK, CHUNK), 1)
    tri = jnp.where(c < r, 1.0, 0.0)
    pos_chunks = []
    acc = jnp.zeros((1, E), jnp.float32)
    for ci in range(NCH):
        ind_c = ind[ci * CHUNK:(ci + 1) * CHUNK, :]
        within = jnp.dot(tri, ind_c, preferred_element_type=jnp.float32)
        pos_chunks.append(within + acc)
        acc = acc + jnp.sum(ind_c, axis=0, keepdims=True)
    pos = jnp.concatenate(pos_chunks, axis=0)
    cnt = acc

    ntiles = (cnt.astype(jnp.int32) + (TM - 1)) // TM
    er = jax.lax.broadcasted_iota(jnp.int32, (E, E), 0)
    ec = jax.lax.broadcasted_iota(jnp.int32, (E, E), 1)
    tri8 = jnp.where(er < ec, 1.0, 0.0)
    tstart = jnp.dot(ntiles.astype(jnp.float32), tri8,
                     preferred_element_type=jnp.float32).astype(jnp.int32)
    offpad = (tstart * TM).astype(jnp.float32)

    a = offpad + pos
    d1 = jnp.sum(jnp.where(sel1, a, 0.0), axis=1)
    d2 = jnp.sum(jnp.where(sel2, a, 0.0), axis=1)
    d1_ref[...] = d1.astype(jnp.int32)[:, None]
    d2_ref[...] = d2.astype(jnp.int32)[:, None]

    ends = tstart.astype(jnp.float32) + ntiles.astype(jnp.float32)
    ident = jnp.where(er == ec, 1.0, 0.0)
    ends_sub = jax.lax.dot_general(ident, ends, (((1,), (1,)), ((), ())),
                                   preferred_element_type=jnp.float32)
    jl = jax.lax.broadcasted_iota(jnp.int32, (E, NT), 1).astype(jnp.float32)
    eid = jnp.minimum(jnp.sum(jnp.where(jl >= ends_sub, 1.0, 0.0), axis=0),
                      float(E - 1))
    used = jnp.sum(ntiles.astype(jnp.float32), axis=1, keepdims=True)
    meta_ref[...] = jnp.concatenate([eid[None, :], used],
                                    axis=1).astype(jnp.int32)


_NC, _NS = 2, 16
_NW = _NC * _NS
_RPW = (2 * T) // _NW
_RC = 128


def _sc_scatter_rows(x_hbm, idx_hbm, out_hbm, idx_v, rows_v, sem):
    wid = lax.axis_index("s") * _NC + lax.axis_index("c")
    for k in range(_RPW // _RC):
        base_f = wid * _RPW + k * _RC
        base_src = jnp.where(base_f >= T, base_f - T, base_f)
        pltpu.sync_copy(idx_hbm.at[pl.ds(base_f, _RC)], idx_v)
        pltpu.sync_copy(x_hbm.at[pl.ds(base_src, _RC)], rows_v)
        pltpu.async_copy(rows_v, out_hbm.at[idx_v], sem).wait()


def _dispatch_scatter(xt, dcat):
    sc = pl.kernel(
        _sc_scatter_rows,
        out_type=jax.ShapeDtypeStruct((P, D), jnp.float32),
        mesh=plsc.VectorSubcoreMesh(core_axis_name="c", subcore_axis_name="s"),
        scratch_types=[
            pltpu.VMEM((_RC,), jnp.int32),
            pltpu.VMEM((_RC, D), jnp.float32),
            pltpu.SemaphoreType.DMA,
        ],
    )
    return sc(xt, dcat)


def _gmm_body(meta_ref, xs_ref, w1_ref, b1_ref, w2_ref, b2_ref,
              ys_ref, w1b_ref, w2b_ref):
    j = pl.program_id(0)
    used = meta_ref[NT]

    jm = jnp.maximum(j - 1, 0)
    changed = jnp.logical_or(j == 0, meta_ref[j] != meta_ref[jm])

    @pl.when(jnp.logical_and(changed, j < used))
    def _():
        w1b_ref[...] = w1_ref[0].astype(jnp.bfloat16)
        w2b_ref[...] = w2_ref[0].astype(jnp.bfloat16)

    @pl.when(j < used)
    def _():
        xb = xs_ref[...].astype(jnp.bfloat16)
        h = jnp.dot(xb, w1b_ref[...], preferred_element_type=jnp.float32)
        h = jnp.maximum(h + b1_ref[0], 0.0)
        o = jnp.dot(h.astype(jnp.bfloat16), w2b_ref[...],
                    preferred_element_type=jnp.float32)
        ys_ref[...] = o + b2_ref[0]


def _combine_body(x_ref, y1_ref, y2_ref, wa_ref, wb_ref, lng_ref, lnb_ref,
                  out_ref):
    y = (x_ref[...] + wa_ref[...] * y1_ref[...] + wb_ref[...] * y2_ref[...])
    mu = jnp.mean(y, axis=-1, keepdims=True)
    var = jnp.mean((y - mu) ** 2, axis=-1, keepdims=True)
    out_ref[...] = ((y - mu) * jax.lax.rsqrt(var + 1e-5) * lng_ref[...]
                    + lnb_ref[...])


def kernel(x, gate_W, gate_b, W1, b1, W2, b2, ln_g, ln_b):
    nkey = jax.random.key(42)
    gumbel = -jnp.log(jax.random.exponential(nkey, (B, S, E), dtype=jnp.float32))

    xt = x.reshape(T, D)
    gum = gumbel.reshape(T, E)

    d1, d2, wa, wb, meta2d = pl.pallas_call(
        _route_body,
        grid=(1,),
        in_specs=[
            pl.BlockSpec((T, D), lambda i: (0, 0)),
            pl.BlockSpec((D, E), lambda i: (0, 0)),
            pl.BlockSpec((1, E), lambda i: (0, 0)),
            pl.BlockSpec((T, E), lambda i: (0, 0)),
        ],
        out_specs=[
            pl.BlockSpec((T, 1), lambda i: (0, 0)),
            pl.BlockSpec((T, 1), lambda i: (0, 0)),
            pl.BlockSpec((T, 1), lambda i: (0, 0)),
            pl.BlockSpec((T, 1), lambda i: (0, 0)),
            pl.BlockSpec((1, NT + 1), lambda i: (0, 0)),
        ],
        out_shape=[
            jax.ShapeDtypeStruct((T, 1), jnp.int32),
            jax.ShapeDtypeStruct((T, 1), jnp.int32),
            jax.ShapeDtypeStruct((T, 1), jnp.float32),
            jax.ShapeDtypeStruct((T, 1), jnp.float32),
            jax.ShapeDtypeStruct((1, NT + 1), jnp.int32),
        ],
    )(xt, gate_W, gate_b.reshape(1, E), gum)
    meta = meta2d.reshape(NT + 1)

    dcat = jnp.concatenate([d1[:, 0], d2[:, 0]])
    xs = _dispatch_scatter(xt, dcat)

    ys = pl.pallas_call(
        _gmm_body,
        grid_spec=pltpu.PrefetchScalarGridSpec(
            num_scalar_prefetch=1,
            grid=(NT,),
            in_specs=[
                pl.BlockSpec((TM, D), lambda j, m: (j, 0)),
                pl.BlockSpec((1, D, H), lambda j, m: (m[j], 0, 0)),
                pl.BlockSpec((1, 1, H), lambda j, m: (m[j], 0, 0)),
                pl.BlockSpec((1, H, D), lambda j, m: (m[j], 0, 0)),
                pl.BlockSpec((1, 1, D), lambda j, m: (m[j], 0, 0)),
            ],
            out_specs=pl.BlockSpec((TM, D), lambda j, m: (j, 0)),
            scratch_shapes=[
                pltpu.VMEM((D, H), jnp.bfloat16),
                pltpu.VMEM((H, D), jnp.bfloat16),
            ],
        ),
        out_shape=jax.ShapeDtypeStruct((P, D), jnp.float32),
    )(meta, xs, W1, b1.reshape(E, 1, H), W2, b2.reshape(E, 1, D))

    ycat = ys[dcat]
    y1 = ycat[:T]
    y2 = ycat[T:]

    out = pl.pallas_call(
        _combine_body,
        grid=(T // CB,),
        in_specs=[
            pl.BlockSpec((CB, D), lambda t: (t, 0)),
            pl.BlockSpec((CB, D), lambda t: (t, 0)),
            pl.BlockSpec((CB, D), lambda t: (t, 0)),
            pl.BlockSpec((CB, 1), lambda t: (t, 0)),
            pl.BlockSpec((CB, 1), lambda t: (t, 0)),
            pl.BlockSpec((1, D), lambda t: (0, 0)),
            pl.BlockSpec((1, D), lambda t: (0, 0)),
        ],
        out_specs=pl.BlockSpec((CB, D), lambda t: (t, 0)),
        out_shape=jax.ShapeDtypeStruct((T, D), jnp.float32),
    )(xt, y1, y2, wa, wb, ln_g.reshape(1, D), ln_b.reshape(1, D))
    return out.reshape(B, S, D)


# speedup vs baseline: 1.0469x; 1.0469x over previous
import functools

import jax
import jax.numpy as jnp
from jax import lax
from jax.experimental import pallas as pl
from jax.experimental.pallas import tpu as pltpu
from jax.experimental.pallas import tpu_sc as plsc

B, S, D = 2, 2048, 768
E, H, TOP_K, TAU = 8, 2048, 2, 1.0
T = B * S
TM = 512
P = 2 * T + E * TM
NT = P // TM
CHUNK = 512
NCH = T // CHUNK
CB = 1024


def _route_body(x_ref, gw_ref, gb_ref, gum_ref,
                d1_ref, d2_ref, wa_ref, wb_ref, meta_ref):
    x = x_ref[...]
    logits = jnp.dot(x, gw_ref[...], preferred_element_type=jnp.float32)
    noisy = logits + gb_ref[...] + gum_ref[...]

    v1 = jnp.max(noisy, axis=-1)
    i1 = jnp.argmax(noisy, axis=-1)
    cols = jax.lax.broadcasted_iota(jnp.int32, noisy.shape, 1)
    masked = jnp.where(cols == i1[:, None], -jnp.inf, noisy)
    v2 = jnp.max(masked, axis=-1)
    i2 = jnp.argmax(masked, axis=-1)
    tso = jnp.exp(v2 - v1)
    wa_ref[...] = (1.0 / (1.0 + tso))[:, None]
    wb_ref[...] = (tso / (1.0 + tso))[:, None]

    sel1 = cols == i1[:, None]
    sel2 = cols == i2[:, None]
    ind = jnp.where(jnp.logical_or(sel1, sel2), 1.0, 0.0)

    r = jax.lax.broadcasted_iota(jnp.int32, (CHUNK, CHUNK), 0)
    c = jax.lax.broadcasted_iota(jnp.int32, (CHUNK, CHUNK), 1)
    tri = jnp.where(c < r, 1.0, 0.0)
    pos_chunks = []
    acc = jnp.zeros((1, E), jnp.float32)
    for ci in range(NCH):
        ind_c = ind[ci * CHUNK:(ci + 1) * CHUNK, :]
        within = jnp.dot(tri, ind_c, preferred_element_type=jnp.float32)
        pos_chunks.append(within + acc)
        acc = acc + jnp.sum(ind_c, axis=0, keepdims=True)
    pos = jnp.concatenate(pos_chunks, axis=0)
    cnt = acc

    ntiles = (cnt.astype(jnp.int32) + (TM - 1)) // TM
    er = jax.lax.broadcasted_iota(jnp.int32, (E, E), 0)
    ec = jax.lax.broadcasted_iota(jnp.int32, (E, E), 1)
    tri8 = jnp.where(er < ec, 1.0, 0.0)
    tstart = jnp.dot(ntiles.astype(jnp.float32), tri8,
                     preferred_element_type=jnp.float32).astype(jnp.int32)
    offpad = (tstart * TM).astype(jnp.float32)

    a = offpad + pos
    d1 = jnp.sum(jnp.where(sel1, a, 0.0), axis=1)
    d2 = jnp.sum(jnp.where(sel2, a, 0.0), axis=1)
    d1_ref[...] = d1.astype(jnp.int32)[:, None]
    d2_ref[...] = d2.astype(jnp.int32)[:, None]

    ends = tstart.astype(jnp.float32) + ntiles.astype(jnp.float32)
    ident = jnp.where(er == ec, 1.0, 0.0)
    ends_sub = jax.lax.dot_general(ident, ends, (((1,), (1,)), ((), ())),
                                   preferred_element_type=jnp.float32)
    jl = jax.lax.broadcasted_iota(jnp.int32, (E, NT), 1).astype(jnp.float32)
    eid = jnp.minimum(jnp.sum(jnp.where(jl >= ends_sub, 1.0, 0.0), axis=0),
                      float(E - 1))
    used = jnp.sum(ntiles.astype(jnp.float32), axis=1, keepdims=True)
    meta_ref[...] = jnp.concatenate([eid[None, :], used],
                                    axis=1).astype(jnp.int32)


_NC, _NS = 2, 16
_NW = _NC * _NS
_RPW = (2 * T) // _NW
_RC = 128


def _sc_scatter_rows(x_hbm, idx_hbm, out_hbm, idx_v, rows_v, sem):
    wid = lax.axis_index("s") * _NC + lax.axis_index("c")
    for k in range(_RPW // _RC):
        base_f = wid * _RPW + k * _RC
        base_src = jnp.where(base_f >= T, base_f - T, base_f)
        pltpu.sync_copy(idx_hbm.at[pl.ds(base_f, _RC)], idx_v)
        pltpu.sync_copy(x_hbm.at[pl.ds(base_src, _RC)], rows_v)
        pltpu.async_copy(rows_v, out_hbm.at[idx_v], sem).wait()


def _dispatch_scatter(xt, dcat):
    sc = pl.kernel(
        _sc_scatter_rows,
        out_type=jax.ShapeDtypeStruct((P, D), jnp.float32),
        mesh=plsc.VectorSubcoreMesh(core_axis_name="c", subcore_axis_name="s"),
        scratch_types=[
            pltpu.VMEM((_RC,), jnp.int32),
            pltpu.VMEM((_RC, D), jnp.float32),
            pltpu.SemaphoreType.DMA,
        ],
    )
    return sc(xt, dcat)


def _gmm_body(meta_ref, xs_ref, w1_ref, b1_ref, w2_ref, b2_ref,
              ys_ref, w1b_ref, w2b_ref):
    j = pl.program_id(0)
    used = meta_ref[NT]

    jm = jnp.maximum(j - 1, 0)
    changed = jnp.logical_or(j == 0, meta_ref[j] != meta_ref[jm])

    @pl.when(jnp.logical_and(changed, j < used))
    def _():
        w1b_ref[...] = w1_ref[0].astype(jnp.bfloat16)
        w2b_ref[...] = w2_ref[0].astype(jnp.bfloat16)

    @pl.when(j < used)
    def _():
        xb = xs_ref[...].astype(jnp.bfloat16)
        h = jnp.dot(xb, w1b_ref[...], preferred_element_type=jnp.float32)
        h = jnp.maximum(h + b1_ref[0], 0.0)
        o = jnp.dot(h.astype(jnp.bfloat16), w2b_ref[...],
                    preferred_element_type=jnp.float32)
        ys_ref[...] = o + b2_ref[0]


def _combine_body(x_ref, y1_ref, y2_ref, wa_ref, wb_ref, lng_ref, lnb_ref,
                  out_ref):
    y = (x_ref[...] + wa_ref[...] * y1_ref[...] + wb_ref[...] * y2_ref[...])
    mu = jnp.mean(y, axis=-1, keepdims=True)
    var = jnp.mean((y - mu) ** 2, axis=-1, keepdims=True)
    out_ref[...] = ((y - mu) * jax.lax.rsqrt(var + 1e-5) * lng_ref[...]
                    + lnb_ref[...])


def kernel(x, gate_W, gate_b, W1, b1, W2, b2, ln_g, ln_b):
    nkey = jax.random.key(42)
    gumbel = -jnp.log(jax.random.exponential(nkey, (B, S, E), dtype=jnp.float32))

    xt = x.reshape(T, D)
    gum = gumbel.reshape(T, E)

    d1, d2, wa, wb, meta2d = pl.pallas_call(
        _route_body,
        grid=(1,),
        in_specs=[
            pl.BlockSpec((T, D), lambda i: (0, 0)),
            pl.BlockSpec((D, E), lambda i: (0, 0)),
            pl.BlockSpec((1, E), lambda i: (0, 0)),
            pl.BlockSpec((T, E), lambda i: (0, 0)),
        ],
        out_specs=[
            pl.BlockSpec((T, 1), lambda i: (0, 0)),
            pl.BlockSpec((T, 1), lambda i: (0, 0)),
            pl.BlockSpec((T, 1), lambda i: (0, 0)),
            pl.BlockSpec((T, 1), lambda i: (0, 0)),
            pl.BlockSpec((1, NT + 1), lambda i: (0, 0)),
        ],
        out_shape=[
            jax.ShapeDtypeStruct((T, 1), jnp.int32),
            jax.ShapeDtypeStruct((T, 1), jnp.int32),
            jax.ShapeDtypeStruct((T, 1), jnp.float32),
            jax.ShapeDtypeStruct((T, 1), jnp.float32),
            jax.ShapeDtypeStruct((1, NT + 1), jnp.int32),
        ],
    )(xt, gate_W, gate_b.reshape(1, E), gum)
    meta = meta2d.reshape(NT + 1)

    dcat = jnp.concatenate([d1[:, 0], d2[:, 0]])
    xs = _dispatch_scatter(xt, dcat)

    ys = pl.pallas_call(
        _gmm_body,
        grid_spec=pltpu.PrefetchScalarGridSpec(
            num_scalar_prefetch=1,
            grid=(NT,),
            in_specs=[
                pl.BlockSpec((TM, D), lambda j, m: (j, 0)),
                pl.BlockSpec((1, D, H), lambda j, m: (m[j], 0, 0)),
                pl.BlockSpec((1, 1, H), lambda j, m: (m[j], 0, 0)),
                pl.BlockSpec((1, H, D), lambda j, m: (m[j], 0, 0)),
                pl.BlockSpec((1, 1, D), lambda j, m: (m[j], 0, 0)),
            ],
            out_specs=pl.BlockSpec((TM, D), lambda j, m: (j, 0)),
            scratch_shapes=[
                pltpu.VMEM((D, H), jnp.bfloat16),
                pltpu.VMEM((H, D), jnp.bfloat16),
            ],
        ),
        out_shape=jax.ShapeDtypeStruct((P, D), jnp.float32),
    )(meta, xs, W1, b1.reshape(E, 1, H), W2, b2.reshape(E, 1, D))

    y1 = ys[d1[:, 0]]
    y2 = ys[d2[:, 0]]

    out = pl.pallas_call(
        _combine_body,
        grid=(T // CB,),
        in_specs=[
            pl.BlockSpec((CB, D), lambda t: (t, 0)),
            pl.BlockSpec((CB, D), lambda t: (t, 0)),
            pl.BlockSpec((CB, D), lambda t: (t, 0)),
            pl.BlockSpec((CB, 1), lambda t: (t, 0)),
            pl.BlockSpec((CB, 1), lambda t: (t, 0)),
            pl.BlockSpec((1, D), lambda t: (0, 0)),
            pl.BlockSpec((1, D), lambda t: (0, 0)),
        ],
        out_specs=pl.BlockSpec((CB, D), lambda t: (t, 0)),
        out_shape=jax.ShapeDtypeStruct((T, D), jnp.float32),
    )(xt, y1, y2, wa, wb, ln_g.reshape(1, D), ln_b.reshape(1, D))
    return out.reshape(B, S, D)
